# baseline (speedup 1.0000x reference)
import functools

import jax
import jax.numpy as jnp
from jax import lax
from jax.experimental import pallas as pl
from jax.experimental.pallas import tpu as pltpu
from jax.experimental.pallas import tpu_sc as plsc

_B, _D, _P, _FD, _K = 4096, 3, 256, 512, 10
_R = _B * _D
_ROWW = 2 * _P
_NC, _NS, _L = 2, 16, 16
_NW = _NC * _NS
_RPW = _R // _NW
_G = _RPW // _L
_BUFW = _L * _ROWW
_NEG = -3.4e38


def _mse_body(f1_ref, f2_ref, out_ref):
    d = f1_ref[...] - f2_ref[...]
    out_ref[...] = jnp.sum(d * d)[None, None]


def _feature_sq_sum(f1, f2):
    return pl.pallas_call(
        _mse_body,
        out_shape=jax.ShapeDtypeStruct((1, 1), jnp.float32),
    )(f1, f2)


def _sc_body(d1_hbm, d2_hbm, out_hbm, buf1, buf2, p1t, p2t, m1, a1, m2, a2,
             accv):
    wid = lax.axis_index("s") * _NC + lax.axis_index("c")
    lvec = lax.iota(jnp.int32, _L)
    hvec = lvec >> 1
    evenmask = (lvec & 1) == 0
    negv = jnp.full((_L,), _NEG, jnp.float32)
    zeroi = jnp.zeros((_L,), jnp.int32)

    def splat(s):
        return jnp.full((_L,), s, jnp.int32)

    def build_pers(buf, pt):
        def bm(i, carry):
            r = i >> 5
            m = i & 31
            s = r * _ROWW + m * 16
            va = buf[pl.ds(s, _L)]
            vb = buf[pl.ds(s + 1, _L)]
            diff = vb - va
            p = splat(m * 8) + hvec
            idx = p * 16 + ((splat(r) + p) & 15)
            plsc.store_scatter(pt, [idx], diff, mask=evenmask)
            return carry
        lax.fori_loop(0, _L * 32, bm, 0)

    def chunk_scan(pt, mref, aref):
        def cb(c, carry):
            def jb(j, ma):
                mx, am = ma
                pv = splat(c * 16 + j)
                idx = pv * 16 + ((lvec + pv) & 15)
                v = plsc.load_gather(pt, [idx])
                cond = v > mx
                return (jnp.where(cond, v, mx), jnp.where(cond, pv, am))
            mx, am = lax.fori_loop(0, 16, jb, (negv, zeroi))
            mref[pl.ds(c * 16, _L)] = mx
            aref[pl.ds(c * 16, _L)] = am
            return carry
        lax.fori_loop(0, 16, cb, 0)

    def select_one(pt, mref, aref, buf):
        def gm(c, mc):
            mb, cb = mc
            v = mref[pl.ds(c * 16, _L)]
            cond = v > mb
            return (jnp.where(cond, v, mb), jnp.where(cond, splat(c), cb))
        mb, cb = lax.fori_loop(0, 16, gm, (negv, zeroi))
        pstar = plsc.load_gather(aref, [cb * 16 + lvec])
        rbase = lvec * _ROWW + 2 * pstar
        bsel = plsc.load_gather(buf, [rbase])
        dsel = plsc.load_gather(buf, [rbase + 1])
        plsc.store_scatter(pt, [pstar * 16 + ((lvec + pstar) & 15)], negv)

        def rb(j, ma):
            mx, am = ma
            pv = cb * 16 + splat(j)
            idx = pv * 16 + ((lvec + pv) & 15)
            v = plsc.load_gather(pt, [idx])
            cond = v > mx
            return (jnp.where(cond, v, mx), jnp.where(cond, pv, am))
        mn, an = lax.fori_loop(0, 16, rb, (negv, zeroi))
        plsc.store_scatter(mref, [cb * 16 + lvec], mn)
        plsc.store_scatter(aref, [cb * 16 + lvec], an)
        return bsel, dsel

    def group_body(g, acc):
        base = (wid * _G + g) * _BUFW
        pltpu.sync_copy(d1_hbm.at[pl.ds(base, _BUFW)], buf1.at[pl.ds(0, _BUFW)])
        pltpu.sync_copy(d2_hbm.at[pl.ds(base, _BUFW)], buf2.at[pl.ds(0, _BUFW)])
        build_pers(buf1, p1t)
        build_pers(buf2, p2t)
        chunk_scan(p1t, m1, a1)
        chunk_scan(p2t, m2, a2)

        def kstep(kk, acc2):
            b1, dd1 = select_one(p1t, m1, a1, buf1)
            b2, dd2 = select_one(p2t, m2, a2, buf2)
            db = b1 - b2
            dd = dd1 - dd2
            return acc2 + db * db + dd * dd
        return lax.fori_loop(0, _K, kstep, acc)

    acc = lax.fori_loop(0, _G, group_body, jnp.zeros((_L,), jnp.float32))
    accv[...] = acc
    pltpu.sync_copy(accv, out_hbm.at[wid])


def _sc_wasserstein(d1_flat, d2_flat):
    mesh = plsc.VectorSubcoreMesh(core_axis_name="c", subcore_axis_name="s",
                                  num_cores=_NC, num_subcores=_NS)
    run = functools.partial(
        pl.kernel,
        out_type=jax.ShapeDtypeStruct((_NW, _L), jnp.float32),
        mesh=mesh,
        compiler_params=pltpu.CompilerParams(needs_layout_passes=False),
        scratch_types=[
            pltpu.VMEM((_BUFW + 8,), jnp.float32),
            pltpu.VMEM((_BUFW + 8,), jnp.float32),
            pltpu.VMEM((_P * _L,), jnp.float32),
            pltpu.VMEM((_P * _L,), jnp.float32),
            pltpu.VMEM((16 * _L,), jnp.float32),
            pltpu.VMEM((16 * _L,), jnp.int32),
            pltpu.VMEM((16 * _L,), jnp.float32),
            pltpu.VMEM((16 * _L,), jnp.int32),
            pltpu.VMEM((_L,), jnp.float32),
        ],
    )(_sc_body)
    return run(d1_flat, d2_flat)


def kernel(features1, features2, diagrams1, diagrams2):
    f_sq = _feature_sq_sum(features1, features2)
    parts = _sc_wasserstein(diagrams1.reshape(-1), diagrams2.reshape(-1))
    feat = f_sq[0, 0] / (_B * _FD)
    wass = jnp.sum(parts) / (_R * _K)
    return feat + wass


# speedup vs baseline: 88.5547x; 88.5547x over previous
import functools

import jax
import jax.numpy as jnp
from jax import lax
from jax.experimental import pallas as pl
from jax.experimental.pallas import tpu as pltpu
from jax.experimental.pallas import tpu_sc as plsc

_B, _D, _P, _FD, _K = 4096, 3, 256, 512, 10
_R = _B * _D
_ROWW = 2 * _P
_NC, _NS, _L = 2, 16, 16
_NW = _NC * _NS
_RPW = _R // _NW
_G = _RPW // _L
_BUFW = _L * _ROWW
_NEG = -3.4e38


def _mse_body(f1_ref, f2_ref, out_ref):
    d = f1_ref[...] - f2_ref[...]
    out_ref[...] = jnp.sum(d * d)[None, None]


def _feature_sq_sum(f1, f2):
    return pl.pallas_call(
        _mse_body,
        out_shape=jax.ShapeDtypeStruct((1, 1), jnp.float32),
    )(f1, f2)


def _sc_body(d1_hbm, d2_hbm, out_hbm, b1a, b2a, b1b, b2b, p1t, p2t,
             m1, a1, m2, a2, accv, sema, semb):
    wid = lax.axis_index("s") * _NC + lax.axis_index("c")
    lvec = lax.iota(jnp.int32, _L)
    negv = jnp.full((_L,), _NEG, jnp.float32)
    zeroi = jnp.zeros((_L,), jnp.int32)
    mask0 = lvec == 0

    def splat(s):
        return jnp.full((_L,), s, jnp.int32)

    def argmax_tree(vals, idxs):
        vals, idxs = list(vals), list(idxs)
        while len(vals) > 1:
            nv, ni = [], []
            for a in range(0, len(vals), 2):
                cond = vals[a + 1] > vals[a]
                nv.append(jnp.where(cond, vals[a + 1], vals[a]))
                ni.append(jnp.where(cond, idxs[a + 1], idxs[a]))
            vals, idxs = nv, ni
        return vals[0], idxs[0]

    kvec = [j * 16 + ((lvec + j) & 15) for j in range(16)]

    def build_pers(buf, pt, mref, aref):
        def rb(r, c0):
            skew = (splat(r) + lvec) & 15
            addr0 = lvec * 16 + skew
            bvs, dvs = [], []
            for i in range(16):
                s = r * _ROWW + (i >> 3) * 256 + (i & 7) * 16
                bvs.append(buf[pl.ds(s, _L)])
                dvs.append(buf[pl.ds(s + 128, _L)])
            for i in range(16):
                diff = dvs[i] - bvs[i]
                plsc.store_scatter(pt, [addr0 + i * 256], diff)
                cm = jnp.max(diff)
                am = plsc.all_reduce_ffs(diff == cm) + (i * 16)
                cr = splat(i * 16 + r)
                plsc.store_scatter(mref, [cr], jnp.full((_L,), cm), mask=mask0)
                plsc.store_scatter(aref, [cr], am, mask=mask0)
            return c0
        lax.fori_loop(0, _L, rb, 0)

    def select_one(pt, mref, aref, buf, do_maint):
        vals = [mref[pl.ds(c * 16, _L)] for c in range(16)]
        idxs = [splat(c) for c in range(16)]
        mb_, cb_ = argmax_tree(vals, idxs)
        cb16 = cb_ * 16
        pstar = plsc.load_gather(aref, [cb16 + lvec])
        dbase = lvec * _ROWW + (pstar >> 7) * 256 + (pstar & 127)
        bsel = plsc.load_gather(buf, [dbase])
        dsel = plsc.load_gather(buf, [dbase + 128])

        if do_maint:
            tidx = pstar * 16 + ((lvec + pstar) & 15)
            plsc.store_scatter(pt, [tidx], negv)
            base16 = cb16 * 16
            vals2, idxs2 = [], []
            for j in range(16):
                vals2.append(plsc.load_gather(pt, [base16 + kvec[j]]))
                idxs2.append(cb16 + j)
            mn, an = argmax_tree(vals2, idxs2)
            plsc.store_scatter(mref, [cb16 + lvec], mn)
            plsc.store_scatter(aref, [cb16 + lvec], an)
        return bsel, dsel

    def start(g, b1, b2, sem):
        base = (wid * _G + g) * _BUFW
        pltpu.async_copy(d1_hbm.at[pl.ds(base, _BUFW)],
                         b1.at[pl.ds(0, _BUFW)], sem)
        pltpu.async_copy(d2_hbm.at[pl.ds(base, _BUFW)],
                         b2.at[pl.ds(0, _BUFW)], sem)

    def wait(b1, b2, sem):
        base = wid * _G * _BUFW
        pltpu.make_async_copy(d1_hbm.at[pl.ds(base, _BUFW)],
                              b1.at[pl.ds(0, _BUFW)], sem).wait()
        pltpu.make_async_copy(d2_hbm.at[pl.ds(base, _BUFW)],
                              b2.at[pl.ds(0, _BUFW)], sem).wait()

    def compute(b1, b2, acc):
        build_pers(b1, p1t, m1, a1)
        build_pers(b2, p2t, m2, a2)

        def kstep(kk, acc2):
            s1, t1 = select_one(p1t, m1, a1, b1, True)
            s2, t2 = select_one(p2t, m2, a2, b2, True)
            db = s1 - s2
            dd = t1 - t2
            return acc2 + db * db + dd * dd
        acc = lax.fori_loop(0, _K - 1, kstep, acc)
        s1, t1 = select_one(p1t, m1, a1, b1, False)
        s2, t2 = select_one(p2t, m2, a2, b2, False)
        db = s1 - s2
        dd = t1 - t2
        return acc + db * db + dd * dd

    start(0, b1a, b2a, sema)

    def pair(i, acc):
        g = i * 2
        start(g + 1, b1b, b2b, semb)
        wait(b1a, b2a, sema)
        acc = compute(b1a, b2a, acc)
        start(lax.rem(g + 2, _G), b1a, b2a, sema)
        wait(b1b, b2b, semb)
        return compute(b1b, b2b, acc)

    acc = lax.fori_loop(0, _G // 2, pair, jnp.zeros((_L,), jnp.float32))
    wait(b1a, b2a, sema)
    accv[...] = acc
    pltpu.sync_copy(accv, out_hbm.at[wid])


def _sc_wasserstein(d1_flat, d2_flat):
    mesh = plsc.VectorSubcoreMesh(core_axis_name="c", subcore_axis_name="s",
                                  num_cores=_NC, num_subcores=_NS)
    run = functools.partial(
        pl.kernel,
        out_type=jax.ShapeDtypeStruct((_NW, _L), jnp.float32),
        mesh=mesh,
        compiler_params=pltpu.CompilerParams(needs_layout_passes=False),
        scratch_types=[
            pltpu.VMEM((_BUFW + 8,), jnp.float32),
            pltpu.VMEM((_BUFW + 8,), jnp.float32),
            pltpu.VMEM((_BUFW + 8,), jnp.float32),
            pltpu.VMEM((_BUFW + 8,), jnp.float32),
            pltpu.VMEM((_P * _L,), jnp.float32),
            pltpu.VMEM((_P * _L,), jnp.float32),
            pltpu.VMEM((16 * _L,), jnp.float32),
            pltpu.VMEM((16 * _L,), jnp.int32),
            pltpu.VMEM((16 * _L,), jnp.float32),
            pltpu.VMEM((16 * _L,), jnp.int32),
            pltpu.VMEM((_L,), jnp.float32),
            pltpu.SemaphoreType.DMA,
            pltpu.SemaphoreType.DMA,
        ],
    )(_sc_body)
    return run(d1_flat, d2_flat)


def _to_physical(d):
    return d.reshape(_B, _D, 2, 128, 2).transpose(0, 1, 2, 4, 3).reshape(-1)


def kernel(features1, features2, diagrams1, diagrams2):
    f_sq = _feature_sq_sum(features1, features2)
    parts = _sc_wasserstein(_to_physical(diagrams1), _to_physical(diagrams2))
    feat = f_sq[0, 0] / (_B * _FD)
    wass = jnp.sum(parts) / (_R * _K)
    return feat + wass
